# baseline (speedup 1.0000x reference)
import functools

import jax
import jax.numpy as jnp
from jax.experimental import pallas as pl
from jax.experimental.pallas import tpu as pltpu
from jax.experimental.pallas import tpu_sc as plsc

B, NQ, DQ = 2, 100, 256
DKV, H, W = 192, 224, 224
HEADS, DH = 8, 32
INNER = HEADS * DH
TOPW, WS, FACTOR = 4, 8, 2
HC, WC = H // FACTOR, W // FACTOR
M, N = HC // WS, WC // WS
NWIN = M * N
WS2 = WS * WS
PATCH = DKV * FACTOR * FACTOR
BQ = B * NQ
KLEN = TOPW * WS2

_PREC = jax.lax.Precision.DEFAULT


def _dot(a, b, dims, prec=_PREC):
    return jax.lax.dot_general(a, b, (dims, ((), ())),
                               precision=prec,
                               preferred_element_type=jnp.float32)


def _dot_f(a, b, dims):
    return _dot(a, b, dims, prec=jax.lax.Precision.DEFAULT)


def _ln_rows(x, w, b, eps=1e-5):
    mu = jnp.mean(x, axis=-1, keepdims=True)
    var = jnp.mean((x - mu) ** 2, axis=-1, keepdims=True)
    return (x - mu) * jax.lax.rsqrt(var + eps) * w + b


def _softmax_one(x, axis):
    m = jnp.max(x, axis=axis, keepdims=True)
    ex = jnp.exp(x - m)
    return ex / (1.0 + jnp.sum(ex, axis=axis, keepdims=True))


WCHUNK = 8
PCHUNK = 1568


ICH = 8


def _conv_body(img_ref, wmat_ref, bias_ref, xpix_ref):
    zcol = jnp.zeros((DKV, 1), jnp.float32)
    for r in range(ICH):
        a0 = img_ref[0, :, r, 0, :]
        a1 = img_ref[0, :, r, 1, :]
        a0s = jnp.concatenate([a0[:, 1:], zcol], axis=1)
        a1s = jnp.concatenate([a1[:, 1:], zcol], axis=1)
        lhs = jnp.concatenate([a0, a0s, a1, a1s], axis=0)
        full = _dot(lhs, wmat_ref[...], ((0,), (0,)))
        even = full.reshape(WC, FACTOR, INNER)[:, 0, :]
        xpix_ref[pl.ds(r * WC, WC), :] = even + bias_ref[...]


def _stage1a(image5, wmat, bias):
    return pl.pallas_call(
        _conv_body,
        grid=(B, HC // ICH),
        in_specs=[
            pl.BlockSpec((1, DKV, ICH, FACTOR, W),
                         lambda b, i: (b, 0, i, 0, 0)),
            pl.BlockSpec((PATCH, INNER), lambda b, i: (0, 0)),
            pl.BlockSpec((1, INNER), lambda b, i: (0, 0)),
        ],
        out_specs=pl.BlockSpec((ICH * WC, INNER),
                               lambda b, i: (b * (HC // ICH) + i, 0)),
        out_shape=jax.ShapeDtypeStruct((B * HC * WC, INNER), jnp.float32),
    )(image5, wmat, bias)


def _stage1b_body(xwin_ref, imagew_ref):
    xw = xwin_ref[...].reshape(WCHUNK * WS2, INNER)
    scalei = DKV ** (-0.5)
    x3 = xwin_ref[...]
    cor = jax.lax.dot_general(
        x3, x3, (((2,), (2,)), ((0,), (0,))),
        precision=_PREC, preferred_element_type=jnp.float32) * scalei
    cor = _softmax_one(cor.reshape(WCHUNK * WS2, WS2), axis=-1)
    cor = jnp.sum(cor.reshape(WCHUNK, WS2, WS2), axis=1)
    cor = _softmax_one(cor, axis=-1)
    for wdx in range(WCHUNK):
        imagew_ref[wdx, :] = _dot(
            cor[wdx], xw[wdx * WS2:(wdx + 1) * WS2], ((0,), (0,)))


def _stage1b(xwin):
    nwin_total = B * NWIN
    return pl.pallas_call(
        _stage1b_body,
        grid=(nwin_total // WCHUNK,),
        in_specs=[pl.BlockSpec((WCHUNK, WS2, INNER), lambda i: (i, 0, 0))],
        out_specs=pl.BlockSpec((WCHUNK, INNER), lambda i: (i, 0)),
        out_shape=jax.ShapeDtypeStruct((nwin_total, INNER), jnp.float32),
    )(xwin)


def _stage2_body(query_ref, nw_ref, nb_ref, wq_ref, imagew_ref,
                 q_ref, gidx_ref):
    qn = _ln_rows(query_ref[...], nw_ref[...], nb_ref[...])
    q = _dot(qn, wq_ref[...], ((1,), (1,)))
    q_ref[...] = q
    scale = INNER ** (-0.5)
    ar0 = _dot(q[:NQ], imagew_ref[:NWIN], ((1,), (1,))) * scale
    ar1 = _dot(q[NQ:], imagew_ref[NWIN:], ((1,), (1,))) * scale
    ar = jnp.concatenate([ar0, ar1], axis=0)
    col = jax.lax.broadcasted_iota(jnp.int32, ar.shape, 1)
    base = jax.lax.broadcasted_iota(jnp.int32, (BQ, 1), 0)
    base = jnp.where(base >= NQ, NWIN, 0)
    for t in range(TOPW):
        m = jnp.max(ar, axis=-1, keepdims=True)
        cand = jnp.where(ar >= m, col, jnp.int32(2 ** 30))
        idx = jnp.min(cand, axis=-1, keepdims=True)
        for kq in range(4):
            gidx_ref[:, 4 * t + kq:4 * t + kq + 1] = 4 * (idx + base) + kq
        ar = jnp.where(col == idx, jnp.float32(-1e30), ar)


def _stage2(query2d, normq_w, normq_b, wq, imagew):
    return pl.pallas_call(
        _stage2_body,
        out_shape=[
            jax.ShapeDtypeStruct((BQ, INNER), jnp.float32),
            jax.ShapeDtypeStruct((BQ, 4 * TOPW), jnp.int32),
        ],
    )(query2d, normq_w, normq_b, wq, imagew)


NWORK = 32
QROW = WS2 * INNER // 4
GC = 8
NQH = NQ
QSLOTS_H = NQH * 2 * TOPW * 4 // 2
SPW = 56
QPAD = NWORK * SPW
NCHUNK = SPW // GC


def _sc_gather_body(table_hbm, idx_hbm, out_hbm, idx_v, b0, b1, sems):
    wid = jax.lax.axis_index("s") * 2 + jax.lax.axis_index("c")
    pltpu.sync_copy(idx_hbm.at[wid], idx_v)
    base = wid * SPW
    bufs = (b0, b1)
    gd = [None, None]
    sd = [None, None]
    for j in range(NCHUNK):
        p = j % 2
        if gd[p] is not None:
            sd[p].wait()
        src = table_hbm.at[idx_v.at[0, pl.ds(j * GC, GC)]]
        gd[p] = pltpu.async_copy(src, bufs[p], sems.at[p])
        if j > 0:
            q = (j - 1) % 2
            gd[q].wait()
            sd[q] = pltpu.async_copy(
                bufs[q], out_hbm.at[pl.ds(base + (j - 1) * GC, GC)],
                sems.at[2 + q])
    last = (NCHUNK - 1) % 2
    gd[last].wait()
    pltpu.sync_copy(bufs[last], out_hbm.at[pl.ds(base + (NCHUNK - 1) * GC, GC)])
    sd[1 - last].wait()


def _sc_gather(table, idx3):
    mesh = plsc.VectorSubcoreMesh(core_axis_name="c", subcore_axis_name="s",
                                  num_cores=2, num_subcores=16)
    return pl.kernel(
        _sc_gather_body,
        out_type=jax.ShapeDtypeStruct((QPAD, QROW), jnp.float32),
        mesh=mesh,
        scratch_types=[
            pltpu.VMEM((1, SPW), jnp.int32),
            pltpu.VMEM((GC, QROW), jnp.float32),
            pltpu.VMEM((GC, QROW), jnp.float32),
            pltpu.SemaphoreType.DMA((4,)),
        ],
    )(table, idx3)


def _stage3_body(p_ref, pos_ref, nw_ref, nb_ref, wk_ref, wv_ref, q_ref,
                 out_ref):
    kv = p_ref[...].reshape(KLEN, INNER) + pos_ref[...]
    kv = _ln_rows(kv, nw_ref[...], nb_ref[...])
    k = _dot_f(kv, wk_ref[...], ((1,), (1,)))
    v = _dot_f(kv, wv_ref[...], ((1,), (1,)))
    fi = jax.lax.broadcasted_iota(jnp.int32, (INNER, HEADS), 0)
    hi = jax.lax.broadcasted_iota(jnp.int32, (INNER, HEADS), 1)
    sel = (fi // DH == hi).astype(jnp.float32)
    qv = q_ref[0]
    dots = _dot_f(k * qv, sel, ((1,), (0,))) * (DH ** (-0.5))
    dots = dots - jnp.max(dots, axis=0, keepdims=True)
    ex = jnp.exp(dots)
    attn = ex / jnp.sum(ex, axis=0, keepdims=True)
    attn_e = _dot_f(attn, sel, ((1,), (1,)))
    out_ref[0, :, :] = jnp.sum(v * attn_e, axis=0, keepdims=True)


def _attn_half(props, pos, normkv_w, normkv_b, wk, wv, q3d_h):
    return pl.pallas_call(
        _stage3_body,
        grid=(NQH,),
        in_specs=[
            pl.BlockSpec((16, QROW), lambda i: (i, 0)),
            pl.BlockSpec((KLEN, INNER), lambda i: (0, 0)),
            pl.BlockSpec((1, INNER), lambda i: (0, 0)),
            pl.BlockSpec((1, INNER), lambda i: (0, 0)),
            pl.BlockSpec((INNER, INNER), lambda i: (0, 0)),
            pl.BlockSpec((INNER, INNER), lambda i: (0, 0)),
            pl.BlockSpec((1, 1, INNER), lambda i: (i, 0, 0)),
        ],
        out_specs=pl.BlockSpec((1, 1, INNER), lambda i: (i, 0, 0)),
        out_shape=jax.ShapeDtypeStruct((NQH, 1, INNER), jnp.float32),
    )(props, pos, normkv_w, normkv_b, wk, wv, q3d_h)


def _stage3(gidx, xwin, pos, normkv_w, normkv_b, wk, wv, q3d):
    table = xwin.reshape(B * NWIN * 4, QROW)
    pad = jnp.zeros((QPAD - QSLOTS_H,), jnp.int32)
    outs = []
    props = []
    for c in range(2):
        flat = gidx[c * NQH:(c + 1) * NQH].reshape(QSLOTS_H)
        idx3 = jnp.concatenate([flat, pad]).reshape(NWORK, 1, SPW)
        props.append(_sc_gather(table, idx3))
    for c in range(2):
        outs.append(_attn_half(props[c], pos, normkv_w, normkv_b, wk, wv,
                               q3d[c * NQH:(c + 1) * NQH]))
    return jnp.concatenate(outs, axis=0)


def _stage4_body(att_ref, query_ref, wrec_ref, nw_ref, nb_ref,
                 w1_ref, b1_ref, w2_ref, b2_ref, out_ref):
    out = _dot_f(att_ref[...], wrec_ref[...], ((1,), (1,))) + query_ref[...]
    h = _ln_rows(out, nw_ref[...], nb_ref[...])
    h = jnp.maximum(_dot_f(h, w1_ref[...], ((1,), (1,))) + b1_ref[...], 0.0)
    h = _dot_f(h, w2_ref[...], ((1,), (1,))) + b2_ref[...]
    out_ref[...] = out + h


def _stage4(att2d, query2d, wrec, mlp_norm_w, mlp_norm_b,
            mlp_w1, mlp_b1, mlp_w2, mlp_b2):
    return pl.pallas_call(
        _stage4_body,
        out_shape=jax.ShapeDtypeStruct((BQ, DQ), jnp.float32),
    )(att2d, query2d, wrec, mlp_norm_w, mlp_norm_b,
      mlp_w1, mlp_b1, mlp_w2, mlp_b2)


@jax.jit
def kernel(query, image, conv_w, conv_b, key_pos_embed, normq_w, normq_b,
           normkv_w, normkv_b, Wq, Wk, Wv, Wrec, mlp_norm_w, mlp_norm_b,
           mlp_w1, mlp_b1, mlp_w2, mlp_b2):
    image5 = image.reshape(B, DKV, HC, FACTOR, W)
    wmat = conv_w.transpose(2, 3, 1, 0).reshape(PATCH, INNER)
    xpix = _stage1a(image5, wmat, conv_b.reshape(1, INNER))
    xwin = xpix.reshape(B, WS, M, WS, N, INNER)
    xwin = xwin.transpose(0, 2, 4, 1, 3, 5).reshape(B * NWIN, WS2, INNER)
    imagew = _stage1b(xwin)

    q, gidx = _stage2(query.reshape(BQ, DQ), normq_w.reshape(1, DQ),
                      normq_b.reshape(1, DQ), Wq, imagew)

    att = _stage3(gidx, xwin, key_pos_embed.reshape(KLEN, INNER),
                  normkv_w.reshape(1, INNER), normkv_b.reshape(1, INNER),
                  Wk, Wv, q.reshape(BQ, 1, INNER))

    out = _stage4(att.reshape(BQ, INNER), query.reshape(BQ, DQ), Wrec,
                  mlp_norm_w.reshape(1, DQ), mlp_norm_b.reshape(1, DQ),
                  mlp_w1, mlp_b1.reshape(1, DQ), mlp_w2, mlp_b2.reshape(1, DQ))
    return out.reshape(B, NQ, DQ)


# speedup vs baseline: 1.2372x; 1.2372x over previous
import functools

import jax
import jax.numpy as jnp
from jax.experimental import pallas as pl
from jax.experimental.pallas import tpu as pltpu
from jax.experimental.pallas import tpu_sc as plsc

B, NQ, DQ = 2, 100, 256
DKV, H, W = 192, 224, 224
HEADS, DH = 8, 32
INNER = HEADS * DH
TOPW, WS, FACTOR = 4, 8, 2
HC, WC = H // FACTOR, W // FACTOR
M, N = HC // WS, WC // WS
NWIN = M * N
WS2 = WS * WS
PATCH = DKV * FACTOR * FACTOR
BQ = B * NQ
KLEN = TOPW * WS2

_PREC = jax.lax.Precision.DEFAULT


def _dot(a, b, dims, prec=_PREC):
    return jax.lax.dot_general(a, b, (dims, ((), ())),
                               precision=prec,
                               preferred_element_type=jnp.float32)


def _dot_f(a, b, dims):
    return _dot(a, b, dims, prec=jax.lax.Precision.DEFAULT)


def _ln_rows(x, w, b, eps=1e-5):
    mu = jnp.mean(x, axis=-1, keepdims=True)
    var = jnp.mean((x - mu) ** 2, axis=-1, keepdims=True)
    return (x - mu) * jax.lax.rsqrt(var + eps) * w + b


def _softmax_one(x, axis):
    m = jnp.max(x, axis=axis, keepdims=True)
    ex = jnp.exp(x - m)
    return ex / (1.0 + jnp.sum(ex, axis=axis, keepdims=True))


WCHUNK = 8
PCHUNK = 1568


ICH = 8


def _conv_body(img_ref, wmat_ref, bias_ref, xpix_ref):
    zcol = jnp.zeros((DKV, 1), jnp.float32)
    for r in range(ICH):
        a0 = img_ref[0, :, r, 0, :]
        a1 = img_ref[0, :, r, 1, :]
        a0s = jnp.concatenate([a0[:, 1:], zcol], axis=1)
        a1s = jnp.concatenate([a1[:, 1:], zcol], axis=1)
        lhs = jnp.concatenate([a0, a0s, a1, a1s], axis=0)
        full = _dot(lhs, wmat_ref[...], ((0,), (0,)))
        even = full.reshape(WC, FACTOR, INNER)[:, 0, :]
        xpix_ref[pl.ds(r * WC, WC), :] = even + bias_ref[...]


def _stage1a(image5, wmat, bias):
    return pl.pallas_call(
        _conv_body,
        grid=(B, HC // ICH),
        in_specs=[
            pl.BlockSpec((1, DKV, ICH, FACTOR, W),
                         lambda b, i: (b, 0, i, 0, 0)),
            pl.BlockSpec((PATCH, INNER), lambda b, i: (0, 0)),
            pl.BlockSpec((1, INNER), lambda b, i: (0, 0)),
        ],
        out_specs=pl.BlockSpec((ICH * WC, INNER),
                               lambda b, i: (b * (HC // ICH) + i, 0)),
        out_shape=jax.ShapeDtypeStruct((B * HC * WC, INNER), jnp.float32),
    )(image5, wmat, bias)


def _stage1b_body(xwin_ref, imagew_ref):
    xw = xwin_ref[...].reshape(WCHUNK * WS2, INNER)
    scalei = DKV ** (-0.5)
    x3 = xwin_ref[...]
    cor = jax.lax.dot_general(
        x3, x3, (((2,), (2,)), ((0,), (0,))),
        precision=_PREC, preferred_element_type=jnp.float32) * scalei
    cor = _softmax_one(cor.reshape(WCHUNK * WS2, WS2), axis=-1)
    cor = jnp.sum(cor.reshape(WCHUNK, WS2, WS2), axis=1)
    cor = _softmax_one(cor, axis=-1)
    for wdx in range(WCHUNK):
        imagew_ref[wdx, :] = _dot(
            cor[wdx], xw[wdx * WS2:(wdx + 1) * WS2], ((0,), (0,)))


def _stage1b(xwin):
    nwin_total = B * NWIN
    return pl.pallas_call(
        _stage1b_body,
        grid=(nwin_total // WCHUNK,),
        in_specs=[pl.BlockSpec((WCHUNK, WS2, INNER), lambda i: (i, 0, 0))],
        out_specs=pl.BlockSpec((WCHUNK, INNER), lambda i: (i, 0)),
        out_shape=jax.ShapeDtypeStruct((nwin_total, INNER), jnp.float32),
    )(xwin)


def _stage2_body(query_ref, nw_ref, nb_ref, wq_ref, imagew_ref,
                 q_ref, gidx_ref):
    qn = _ln_rows(query_ref[...], nw_ref[...], nb_ref[...])
    q = _dot(qn, wq_ref[...], ((1,), (1,)))
    q_ref[...] = q
    scale = INNER ** (-0.5)
    ar0 = _dot(q[:NQ], imagew_ref[:NWIN], ((1,), (1,))) * scale
    ar1 = _dot(q[NQ:], imagew_ref[NWIN:], ((1,), (1,))) * scale
    ar = jnp.concatenate([ar0, ar1], axis=0)
    col = jax.lax.broadcasted_iota(jnp.int32, ar.shape, 1)
    base = jax.lax.broadcasted_iota(jnp.int32, (BQ, 1), 0)
    base = jnp.where(base >= NQ, NWIN, 0)
    for t in range(TOPW):
        m = jnp.max(ar, axis=-1, keepdims=True)
        cand = jnp.where(ar >= m, col, jnp.int32(2 ** 30))
        idx = jnp.min(cand, axis=-1, keepdims=True)
        for kq in range(4):
            gidx_ref[:, 4 * t + kq:4 * t + kq + 1] = 4 * (idx + base) + kq
        ar = jnp.where(col == idx, jnp.float32(-1e30), ar)


def _stage2(query2d, normq_w, normq_b, wq, imagew):
    return pl.pallas_call(
        _stage2_body,
        out_shape=[
            jax.ShapeDtypeStruct((BQ, INNER), jnp.float32),
            jax.ShapeDtypeStruct((BQ, 4 * TOPW), jnp.int32),
        ],
    )(query2d, normq_w, normq_b, wq, imagew)


NWORK = 32
QROW = WS2 * INNER // 4
QSLOTS = BQ * TOPW * 4
SPW = 104
QPAD = NWORK * SPW
GC = 8
NCHUNK = SPW // GC


def _sc_gather_body(table_hbm, idx_hbm, out_hbm, idx_v, b0, b1, sems):
    wid = jax.lax.axis_index("s") * 2 + jax.lax.axis_index("c")
    pltpu.sync_copy(idx_hbm.at[wid], idx_v)
    base = wid * SPW
    bufs = (b0, b1)
    gd = [None, None]
    sd = [None, None]
    for j in range(NCHUNK):
        p = j % 2
        if gd[p] is not None:
            sd[p].wait()
        src = table_hbm.at[idx_v.at[0, pl.ds(j * GC, GC)]]
        gd[p] = pltpu.async_copy(src, bufs[p], sems.at[p])
        if j > 0:
            q = (j - 1) % 2
            gd[q].wait()
            sd[q] = pltpu.async_copy(
                bufs[q], out_hbm.at[pl.ds(base + (j - 1) * GC, GC)],
                sems.at[2 + q])
    last = (NCHUNK - 1) % 2
    gd[last].wait()
    pltpu.sync_copy(bufs[last], out_hbm.at[pl.ds(base + (NCHUNK - 1) * GC, GC)])
    sd[1 - last].wait()


def _sc_gather(table, idx3):
    mesh = plsc.VectorSubcoreMesh(core_axis_name="c", subcore_axis_name="s",
                                  num_cores=2, num_subcores=16)
    return pl.kernel(
        _sc_gather_body,
        out_type=jax.ShapeDtypeStruct((QPAD, QROW), jnp.float32),
        mesh=mesh,
        scratch_types=[
            pltpu.VMEM((1, SPW), jnp.int32),
            pltpu.VMEM((GC, QROW), jnp.float32),
            pltpu.VMEM((GC, QROW), jnp.float32),
            pltpu.SemaphoreType.DMA((4,)),
        ],
    )(table, idx3)


QB = 2


def _stage3_body(p_ref, pos_ref, nw_ref, nb_ref, wk_ref, wv_ref, q_ref,
                 out_ref):
    kv = p_ref[...].reshape(QB * KLEN, INNER)
    kv = kv + jnp.concatenate([pos_ref[...]] * QB, axis=0)
    kv = _ln_rows(kv, nw_ref[...], nb_ref[...])
    k = _dot_f(kv, wk_ref[...], ((1,), (1,)))
    v = _dot_f(kv, wv_ref[...], ((1,), (1,)))
    fi = jax.lax.broadcasted_iota(jnp.int32, (INNER, HEADS), 0)
    hi = jax.lax.broadcasted_iota(jnp.int32, (INNER, HEADS), 1)
    sel = (fi // DH == hi).astype(jnp.float32)
    for qq in range(QB):
        ks = k[qq * KLEN:(qq + 1) * KLEN]
        vs = v[qq * KLEN:(qq + 1) * KLEN]
        qv = q_ref[0, qq:qq + 1, :]
        dots = _dot_f(ks * qv, sel, ((1,), (0,))) * (DH ** (-0.5))
        dots = dots - jnp.max(dots, axis=0, keepdims=True)
        ex = jnp.exp(dots)
        attn = ex / jnp.sum(ex, axis=0, keepdims=True)
        attn_e = _dot_f(attn, sel, ((1,), (1,)))
        out_ref[0, qq, :] = jnp.sum(vs * attn_e, axis=0)


def _stage3(gidx, xwin, pos, normkv_w, normkv_b, wk, wv, q3d):
    flat = gidx.reshape(QSLOTS)
    idx3 = jnp.concatenate(
        [flat, jnp.zeros((QPAD - QSLOTS,), jnp.int32)]).reshape(NWORK, 1, SPW)
    props = _sc_gather(xwin.reshape(B * NWIN * 4, QROW), idx3)
    return pl.pallas_call(
        _stage3_body,
        grid=(BQ // QB,),
        in_specs=[
            pl.BlockSpec((QB * 16, QROW), lambda i: (i, 0)),
            pl.BlockSpec((KLEN, INNER), lambda i: (0, 0)),
            pl.BlockSpec((1, INNER), lambda i: (0, 0)),
            pl.BlockSpec((1, INNER), lambda i: (0, 0)),
            pl.BlockSpec((INNER, INNER), lambda i: (0, 0)),
            pl.BlockSpec((INNER, INNER), lambda i: (0, 0)),
            pl.BlockSpec((1, QB, INNER), lambda i: (i, 0, 0)),
        ],
        out_specs=pl.BlockSpec((1, QB, INNER), lambda i: (i, 0, 0)),
        out_shape=jax.ShapeDtypeStruct((BQ // QB, QB, INNER), jnp.float32),
    )(props, pos, normkv_w, normkv_b, wk, wv,
      q3d.reshape(BQ // QB, QB, INNER))


def _stage4_body(att_ref, query_ref, wrec_ref, nw_ref, nb_ref,
                 w1_ref, b1_ref, w2_ref, b2_ref, out_ref):
    out = _dot_f(att_ref[...], wrec_ref[...], ((1,), (1,))) + query_ref[...]
    h = _ln_rows(out, nw_ref[...], nb_ref[...])
    h = jnp.maximum(_dot_f(h, w1_ref[...], ((1,), (1,))) + b1_ref[...], 0.0)
    h = _dot_f(h, w2_ref[...], ((1,), (1,))) + b2_ref[...]
    out_ref[...] = out + h


def _stage4(att2d, query2d, wrec, mlp_norm_w, mlp_norm_b,
            mlp_w1, mlp_b1, mlp_w2, mlp_b2):
    return pl.pallas_call(
        _stage4_body,
        out_shape=jax.ShapeDtypeStruct((BQ, DQ), jnp.float32),
    )(att2d, query2d, wrec, mlp_norm_w, mlp_norm_b,
      mlp_w1, mlp_b1, mlp_w2, mlp_b2)


@jax.jit
def kernel(query, image, conv_w, conv_b, key_pos_embed, normq_w, normq_b,
           normkv_w, normkv_b, Wq, Wk, Wv, Wrec, mlp_norm_w, mlp_norm_b,
           mlp_w1, mlp_b1, mlp_w2, mlp_b2):
    image5 = image.reshape(B, DKV, HC, FACTOR, W)
    wmat = conv_w.transpose(2, 3, 1, 0).reshape(PATCH, INNER)
    xpix = _stage1a(image5, wmat, conv_b.reshape(1, INNER))
    xwin = xpix.reshape(B, WS, M, WS, N, INNER)
    xwin = xwin.transpose(0, 2, 4, 1, 3, 5).reshape(B * NWIN, WS2, INNER)
    imagew = _stage1b(xwin)

    q, gidx = _stage2(query.reshape(BQ, DQ), normq_w.reshape(1, DQ),
                      normq_b.reshape(1, DQ), Wq, imagew)

    att = _stage3(gidx, xwin, key_pos_embed.reshape(KLEN, INNER),
                  normkv_w.reshape(1, INNER), normkv_b.reshape(1, INNER),
                  Wk, Wv, q.reshape(BQ, 1, INNER))

    out = _stage4(att.reshape(BQ, INNER), query.reshape(BQ, DQ), Wrec,
                  mlp_norm_w.reshape(1, DQ), mlp_norm_b.reshape(1, DQ),
                  mlp_w1, mlp_b1.reshape(1, DQ), mlp_w2, mlp_b2.reshape(1, DQ))
    return out.reshape(B, NQ, DQ)
